# baseline (speedup 1.0000x reference)
import functools

import jax
import jax.numpy as jnp
from jax.experimental import pallas as pl
from jax.experimental.pallas import tpu as pltpu


def _fused_kernel(x_ref, t1_ref, t2_ref, sel_ref, w0_ref, w1_ref, w2_ref,
                  cb1_ref, cb2_ref, b0_ref, b1_ref, b2_ref, o_ref,
                  *, bb, ph1, pw1, oh2, ph2, pw2, c2):
    f32 = jnp.float32
    x = x_ref[...]
    oh1 = 2 * ph1
    n1 = pw1 * 2 * 10

    x5 = jnp.concatenate(
        [x[:, kh:kh + oh1, :].reshape(bb * oh1, x.shape[-1])
         for kh in range(5)], axis=-1)
    y = jnp.dot(x5, t1_ref[...], preferred_element_type=f32)
    yv = y.reshape(bb, ph1, 2, n1)
    rm = jnp.maximum(yv[:, :, 0, :], yv[:, :, 1, :])
    m = jnp.maximum(rm[:, :, 0:n1 - 10], rm[:, :, 10:n1])
    bc1 = jnp.concatenate([cb1_ref[:, :c2]] * ph1, axis=-1)
    f1 = jnp.maximum(m + bc1[:, :n1 - 10], 0.0)
    f1 = jnp.concatenate([f1, jnp.zeros((bb, ph1, 10), f32)], axis=-1)

    n2 = oh2 * c2
    y2 = jnp.zeros((bb * oh2, n2), f32)
    for kh in range(5):
        fk = f1[:, kh:kh + oh2, :].reshape(bb * oh2, f1.shape[-1])
        y2 = y2 + jnp.dot(fk, t2_ref[kh], preferred_element_type=f32)
    y2v = y2.reshape(bb, ph2, 2, n2)
    rm2 = jnp.maximum(y2v[:, :, 0, :], y2v[:, :, 1, :])
    m2 = jnp.maximum(rm2[:, :, 0:n2 - c2], rm2[:, :, c2:n2])
    bc2 = jnp.concatenate([cb2_ref[:, :c2]] * (2 * ph2 - 1), axis=-1)
    h = jnp.maximum(m2 + bc2, 0.0)

    hc = jnp.dot(h[:, 0, :], sel_ref[0], preferred_element_type=f32)
    for p in range(1, ph2):
        hc = hc + jnp.dot(h[:, p, :], sel_ref[p], preferred_element_type=f32)
    nk = (((1,), (1,)), ((), ()))
    u = jax.lax.dot_general(hc, w0_ref[...], nk, preferred_element_type=f32)
    u = jnp.maximum(u + b0_ref[...], 0.0)
    v = jnp.dot(u, w1_ref[...], preferred_element_type=f32)
    v = jnp.maximum(v + b1_ref[...], 0.0)
    dim_out = o_ref.shape[-1]
    o_ref[0] = (jnp.dot(v, w2_ref[...], preferred_element_type=f32)
                + b2_ref[...])[:, :dim_out].astype(o_ref.dtype)


def kernel(x, conv1_w, conv1_b_raw, conv2_w, conv2_b_raw, conv1_wmat,
           conv1_b, conv2_wmat, conv2_b, mlp_w0, mlp_b0, mlp_w1, mlp_b1,
           mlp_w2, mlp_b2, lin_w0, lin_b0, lin_w1, lin_b1, lin_w2, lin_b2):
    f32 = jnp.float32
    nt, no = x.shape[:2]
    B = nt * no
    H = x.shape[-1]
    c1 = conv1_w.shape[0]
    c2 = conv2_w.shape[0]
    oh1 = H - 4
    ph1 = oh1 // 2
    oh2 = ph1 - 4
    ph2 = oh2 // 2
    hid1 = lin_w0.shape[0]
    hid2 = lin_w1.shape[0]
    dim_out = lin_w2.shape[0]

    x3 = x.reshape(B, H, H)

    w1m = conv1_w[:, 0]
    e1b = (jnp.arange(H)[:, None, None]
           == jnp.arange(oh1)[None, :, None]
           + jnp.arange(5)[None, None, :]).astype(f32)
    e1h = jnp.broadcast_to(e1b[None], (5, H, oh1, 5))
    t1 = jnp.einsum('hiok,chk->hioc', e1h, w1m).reshape(5 * H, oh1 * c1)

    c2b = ((jnp.arange(ph1)[:, None, None, None, None]
            == jnp.arange(oh2)[None, None, None, :, None]
            + jnp.arange(5)[None, None, None, None, :])
           & (jnp.arange(2 * c1)[None, :, None, None, None]
              == jnp.arange(c1)[None, None, :, None, None])
           ).astype(f32)
    c2q = jnp.broadcast_to(c2b[None], (5,) + c2b.shape)
    t2 = jnp.einsum('hzpcok,dchk->hzpod', c2q, conv2_w
                    ).reshape(5, ph1 * c2, oh2 * c2)

    pw2 = ph2
    nlane = 2 * ph2 * c2 - c2
    nfeat = c2 * ph2 * pw2
    la = jnp.arange(nlane)[None, :, None]
    ja = jnp.arange(nfeat)[None, None, :]
    pa = jnp.arange(ph2)[:, None, None]
    p2 = ph2 * pw2
    sel = ((la % (2 * c2) == ja // p2)
           & (la // (2 * c2) == ja % pw2)
           & ((ja % p2) // pw2 == pa)
           & (la % (2 * c2) < c2)).astype(f32)

    nsplit = 1
    bb = B // nsplit
    zero = lambda i: (0, 0, 0)
    out = pl.pallas_call(
        functools.partial(_fused_kernel, bb=bb, ph1=ph1, pw1=ph1, oh2=oh2,
                          ph2=ph2, pw2=ph2, c2=c2),
        out_shape=jax.ShapeDtypeStruct((nsplit, bb, dim_out), f32),
        grid=(nsplit,),
        in_specs=[
            pl.BlockSpec((bb, H, H), lambda i: (i, 0, 0)),
            pl.BlockSpec(t1.shape, lambda i: (0, 0)),
            pl.BlockSpec(t2.shape, zero),
            pl.BlockSpec(sel.shape, zero),
            pl.BlockSpec(lin_w0.shape, lambda i: (0, 0)),
            pl.BlockSpec(mlp_w1.shape, lambda i: (0, 0)),
            pl.BlockSpec(mlp_w2.shape, lambda i: (0, 0)),
            pl.BlockSpec(conv1_b.shape, lambda i: (0, 0)),
            pl.BlockSpec(conv2_b.shape, lambda i: (0, 0)),
            pl.BlockSpec(mlp_b0.shape, lambda i: (0, 0)),
            pl.BlockSpec(mlp_b1.shape, lambda i: (0, 0)),
            pl.BlockSpec(mlp_b2.shape, lambda i: (0, 0)),
        ],
        out_specs=pl.BlockSpec((1, bb, dim_out), lambda i: (i, 0, 0)),
        compiler_params=pltpu.CompilerParams(
            dimension_semantics=("parallel",),
            vmem_limit_bytes=50 * 1024 * 1024),
    )(x3, t1, t2, sel, lin_w0, mlp_w1, mlp_w2, conv1_b, conv2_b,
      mlp_b0, mlp_b1, mlp_b2)
    return out.reshape(nt, no, dim_out)


# speedup vs baseline: 1.0855x; 1.0855x over previous
import functools

import jax
import jax.numpy as jnp
from jax.experimental import pallas as pl
from jax.experimental.pallas import tpu as pltpu


def _fused_kernel(x_ref, t1_ref, t2_ref, sel_ref, w0_ref, w1_ref, w2_ref,
                  cb1_ref, cb2_ref, b0_ref, b1_ref, b2_ref, o_ref,
                  *, bb, ph1, pw1, oh2, ph2, pw2, c2):
    f32 = jnp.float32
    x = x_ref[...]
    oh1 = 2 * ph1
    n1 = pw1 * 2 * 10

    x5 = jnp.concatenate(
        [x[:, kh:kh + oh1, :].reshape(bb * oh1, x.shape[-1])
         for kh in range(5)], axis=-1)
    y = jnp.dot(x5, t1_ref[...], preferred_element_type=f32)
    yv = y.reshape(bb, ph1, 2, n1)
    rm = jnp.maximum(yv[:, :, 0, :], yv[:, :, 1, :])
    m = jnp.maximum(rm[:, :, 0:n1 - 10], rm[:, :, 10:n1])
    bc1 = jnp.concatenate([cb1_ref[:, :c2]] * ph1, axis=-1)
    f1 = jnp.maximum(m + bc1[:, :n1 - 10], 0.0)
    f1 = jnp.concatenate([f1, jnp.zeros((bb, ph1, 10), f32)], axis=-1)

    n2 = oh2 * c2
    y2 = jnp.zeros((bb * oh2, n2), f32)
    for kh in range(5):
        fk = f1[:, kh:kh + oh2, :].reshape(bb * oh2, f1.shape[-1])
        y2 = y2 + jnp.dot(fk, t2_ref[kh], preferred_element_type=f32)
    y2v = y2.reshape(bb, ph2, 2, n2)
    rm2 = jnp.maximum(y2v[:, :, 0, :], y2v[:, :, 1, :])
    m2 = jnp.maximum(rm2[:, :, 0:n2 - c2], rm2[:, :, c2:n2])
    bc2 = jnp.concatenate([cb2_ref[:, :c2]] * (2 * ph2 - 1), axis=-1)
    h = jnp.maximum(m2 + bc2, 0.0)

    hc = jnp.dot(h[:, 0, :], sel_ref[0], preferred_element_type=f32)
    for p in range(1, ph2):
        hc = hc + jnp.dot(h[:, p, :], sel_ref[p], preferred_element_type=f32)
    nk = (((1,), (1,)), ((), ()))
    u = jax.lax.dot_general(hc, w0_ref[...], nk, preferred_element_type=f32)
    u = jnp.maximum(u + b0_ref[...], 0.0)
    v = jnp.dot(u, w1_ref[...], preferred_element_type=f32)
    v = jnp.maximum(v + b1_ref[...], 0.0)
    dim_out = o_ref.shape[-1]
    o_ref[0] = (jnp.dot(v, w2_ref[...], preferred_element_type=f32)
                + b2_ref[...])[:, :dim_out].astype(o_ref.dtype)


def kernel(x, conv1_w, conv1_b_raw, conv2_w, conv2_b_raw, conv1_wmat,
           conv1_b, conv2_wmat, conv2_b, mlp_w0, mlp_b0, mlp_w1, mlp_b1,
           mlp_w2, mlp_b2, lin_w0, lin_b0, lin_w1, lin_b1, lin_w2, lin_b2):
    f32 = jnp.float32
    nt, no = x.shape[:2]
    B = nt * no
    H = x.shape[-1]
    c1 = conv1_w.shape[0]
    c2 = conv2_w.shape[0]
    oh1 = H - 4
    ph1 = oh1 // 2
    oh2 = ph1 - 4
    ph2 = oh2 // 2
    hid1 = lin_w0.shape[0]
    hid2 = lin_w1.shape[0]
    dim_out = lin_w2.shape[0]

    x3 = x.reshape(B, H, H)

    w1m = conv1_w[:, 0]
    e1 = (jnp.arange(H)[None, :, None]
          == jnp.arange(oh1)[None, None, :] + jnp.arange(5)[:, None, None]
          ).astype(f32)
    t1 = jnp.einsum('kio,chk->hioc', e1, w1m).reshape(5 * H, oh1 * c1)

    c2b = ((jnp.arange(ph1)[:, None, None, None, None]
            == jnp.arange(oh2)[None, None, None, :, None]
            + jnp.arange(5)[None, None, None, None, :])
           & (jnp.arange(2 * c1)[None, :, None, None, None]
              == jnp.arange(c1)[None, None, :, None, None])
           ).astype(f32)
    t2 = jnp.einsum('zpcok,dchk->hzpod', c2b, conv2_w
                    ).reshape(5, ph1 * c2, oh2 * c2)

    pw2 = ph2
    nlane = 2 * ph2 * c2 - c2
    nfeat = c2 * ph2 * pw2
    la = jnp.arange(nlane)[None, :, None]
    ja = jnp.arange(nfeat)[None, None, :]
    pa = jnp.arange(ph2)[:, None, None]
    p2 = ph2 * pw2
    sel = ((la % (2 * c2) == ja // p2)
           & (la // (2 * c2) == ja % pw2)
           & ((ja % p2) // pw2 == pa)
           & (la % (2 * c2) < c2)).astype(f32)

    nsplit = 1
    bb = B // nsplit
    zero = lambda i: (0, 0, 0)
    out = pl.pallas_call(
        functools.partial(_fused_kernel, bb=bb, ph1=ph1, pw1=ph1, oh2=oh2,
                          ph2=ph2, pw2=ph2, c2=c2),
        out_shape=jax.ShapeDtypeStruct((nsplit, bb, dim_out), f32),
        grid=(nsplit,),
        in_specs=[
            pl.BlockSpec((bb, H, H), lambda i: (i, 0, 0)),
            pl.BlockSpec(t1.shape, lambda i: (0, 0)),
            pl.BlockSpec(t2.shape, zero),
            pl.BlockSpec(sel.shape, zero),
            pl.BlockSpec(lin_w0.shape, lambda i: (0, 0)),
            pl.BlockSpec(mlp_w1.shape, lambda i: (0, 0)),
            pl.BlockSpec(mlp_w2.shape, lambda i: (0, 0)),
            pl.BlockSpec(conv1_b.shape, lambda i: (0, 0)),
            pl.BlockSpec(conv2_b.shape, lambda i: (0, 0)),
            pl.BlockSpec(mlp_b0.shape, lambda i: (0, 0)),
            pl.BlockSpec(mlp_b1.shape, lambda i: (0, 0)),
            pl.BlockSpec(mlp_b2.shape, lambda i: (0, 0)),
        ],
        out_specs=pl.BlockSpec((1, bb, dim_out), lambda i: (i, 0, 0)),
        compiler_params=pltpu.CompilerParams(
            dimension_semantics=("parallel",),
            vmem_limit_bytes=50 * 1024 * 1024),
    )(x3, t1, t2, sel, lin_w0, mlp_w1, mlp_w2, conv1_b, conv2_b,
      mlp_b0, mlp_b1, mlp_b2)
    return out.reshape(nt, no, dim_out)
